# baseline (speedup 1.0000x reference)
import functools

import jax
import jax.numpy as jnp
from jax import lax
from jax.experimental import pallas as pl
from jax.experimental.pallas import tpu as pltpu
from jax.experimental.pallas import tpu_sc as plsc

E = 8
TOPK = 2
D = 2048
DFF = 4096
T = 8192
TE = T // E
BF = 256
NF = DFF // BF


def _mlp_body(x_ref, gate_ref, w1_ref, w2_ref, out_ref, xb_ref):
    f = pl.program_id(1)

    @pl.when(f == 0)
    def _():
        xb_ref[...] = x_ref[...].astype(jnp.bfloat16)

    w1b = w1_ref[0].astype(jnp.bfloat16)
    h = lax.dot_general(xb_ref[...], w1b, (((1,), (1,)), ((), ())),
                        preferred_element_type=jnp.float32)
    hb = jax.nn.gelu(h).astype(jnp.bfloat16)
    w2b = w2_ref[0].astype(jnp.bfloat16)
    y = lax.dot_general(hb, w2b, (((1,), (1,)), ((), ())),
                        preferred_element_type=jnp.float32)

    @pl.when(f == 0)
    def _():
        out_ref[...] = y

    @pl.when(f > 0)
    def _():
        out_ref[...] += y

    @pl.when(f == NF - 1)
    def _():
        out_ref[...] = out_ref[...] * gate_ref[...]


def _expert_mlp(x, gate2d, W1, W2):
    return pl.pallas_call(
        _mlp_body,
        grid=(E, NF),
        in_specs=[
            pl.BlockSpec((TE, D), lambda e, f: (e, 0)),
            pl.BlockSpec((TE, 1), lambda e, f: (e, 0)),
            pl.BlockSpec((1, BF, D), lambda e, f: (e, f, 0)),
            pl.BlockSpec((1, D, BF), lambda e, f: (e, 0, f)),
        ],
        out_specs=pl.BlockSpec((TE, D), lambda e, f: (e, 0)),
        out_shape=jax.ShapeDtypeStruct((T, D), jnp.float32),
        scratch_shapes=[pltpu.VMEM((TE, D), jnp.bfloat16)],
        compiler_params=pltpu.CompilerParams(
            dimension_semantics=("parallel", "arbitrary")),
    )(x, gate2d, W1, W2)


NW = 32
IPW = T // NW
TPW = (T // TOPK) // NW
G = 8
NG = TPW // G


@functools.partial(
    pl.kernel,
    out_type=jax.ShapeDtypeStruct((T,), jnp.int32),
    mesh=plsc.VectorSubcoreMesh(core_axis_name="c", subcore_axis_name="s"),
    scratch_types=[
        pltpu.VMEM((IPW,), jnp.int32),
        pltpu.VMEM((2, 128), jnp.int32),
        pltpu.VMEM((2, 128), jnp.int32),
    ],
)
def _build_inv(nidx_hbm, inv_hbm, nidx_v, idx2_v, val2_v):
    w = lax.axis_index("c") * 16 + lax.axis_index("s")
    base = w * IPW
    pltpu.sync_copy(nidx_hbm.at[pl.ds(base, IPW)], nidx_v)
    for i in range(IPW // 16):
        idx2_v[i // 8, pl.ds((i % 8) * 16, 16)] = nidx_v[pl.ds(i * 16, 16)]
        val2_v[i // 8, pl.ds((i % 8) * 16, 16)] = (
            jnp.arange(16, dtype=jnp.int32) + (base + i * 16))
    for j in range(2):
        pltpu.sync_copy(val2_v.at[j], inv_hbm.at[idx2_v.at[j]])


def _combine_body(res_hbm, inv_hbm, out_hbm, myinv_v, idx_g, rows_v,
                  obuf_v, sem):
    w = lax.axis_index("c") * 16 + lax.axis_index("s")
    t0 = w * TPW
    pltpu.sync_copy(inv_hbm.at[pl.ds(2 * t0, 2 * TPW)], myinv_v)
    for g in range(NG):
        idx_g[...] = myinv_v[pl.ds(g * 2 * G, 2 * G)]
        pltpu.async_copy(res_hbm.at[idx_g], rows_v, sem).wait()
        for j in range(G):
            def _add(k, _):
                obuf_v[j, pl.ds(k * 16, 16)] = (
                    rows_v[2 * j, pl.ds(k * 16, 16)]
                    + rows_v[2 * j + 1, pl.ds(k * 16, 16)])
                return 0
            lax.fori_loop(0, D // 16, _add, 0)
        pltpu.sync_copy(obuf_v, out_hbm.at[pl.ds(t0 + g * G, G)])


@functools.partial(
    pl.kernel,
    out_type=jax.ShapeDtypeStruct((T // TOPK, D), jnp.float32),
    mesh=plsc.VectorSubcoreMesh(core_axis_name="c", subcore_axis_name="s"),
    scratch_types=[
        pltpu.VMEM((2 * TPW,), jnp.int32),
        pltpu.VMEM((2 * G,), jnp.int32),
        pltpu.VMEM((2 * G, D), jnp.float32),
        pltpu.VMEM((G, D), jnp.float32),
        pltpu.SemaphoreType.DMA,
    ],
)
def _combine(res_hbm, inv_hbm, out_hbm, myinv_v, idx_g, rows_v,
             obuf_v, sem):
    _combine_body(res_hbm, inv_hbm, out_hbm, myinv_v, idx_g, rows_v,
                  obuf_v, sem)


def kernel(inputs_shard, gate_weight, choosed_experts, new_index, W1, W2):
    gate2d = gate_weight.reshape(T, 1)
    result = _expert_mlp(inputs_shard, gate2d, W1, W2)
    inv = _build_inv(new_index)
    out2 = _combine(result, inv)
    mlp_bias = jnp.zeros((D,), dtype=out2.dtype)
    return (out2, mlp_bias)


# speedup vs baseline: 1.5934x; 1.5934x over previous
import functools

import jax
import jax.numpy as jnp
from jax import lax
from jax.experimental import pallas as pl
from jax.experimental.pallas import tpu as pltpu
from jax.experimental.pallas import tpu_sc as plsc

E = 8
TOPK = 2
D = 2048
DFF = 4096
T = 8192
TE = T // E


def _fc1_body(x_ref, w1_ref, h_ref, xb_ref):
    f = pl.program_id(1)

    @pl.when(f == 0)
    def _():
        xb_ref[...] = x_ref[...].astype(jnp.bfloat16)

    w1b = w1_ref[0].astype(jnp.bfloat16)
    h = lax.dot_general(xb_ref[...], w1b, (((1,), (1,)), ((), ())),
                        preferred_element_type=jnp.float32)
    h_ref[...] = jax.nn.gelu(h).astype(jnp.bfloat16)


def _fc2_body(h_ref, gate_ref, w2_ref, out_ref):
    w2b = w2_ref[0].astype(jnp.bfloat16)
    y = lax.dot_general(h_ref[...], w2b, (((1,), (1,)), ((), ())),
                        preferred_element_type=jnp.float32)
    out_ref[...] = y * gate_ref[...]


BF = 512
NF1 = DFF // BF
BD = 512
ND = D // BD


def _expert_mlp(x, gate2d, W1, W2):
    h = pl.pallas_call(
        _fc1_body,
        grid=(E, NF1),
        in_specs=[
            pl.BlockSpec((TE, D), lambda e, f: (e, 0)),
            pl.BlockSpec((1, BF, D), lambda e, f: (e, f, 0)),
        ],
        out_specs=pl.BlockSpec((TE, BF), lambda e, f: (e, f)),
        out_shape=jax.ShapeDtypeStruct((T, DFF), jnp.bfloat16),
        scratch_shapes=[pltpu.VMEM((TE, D), jnp.bfloat16)],
        compiler_params=pltpu.CompilerParams(
            dimension_semantics=("parallel", "arbitrary")),
    )(x, W1)
    return pl.pallas_call(
        _fc2_body,
        grid=(E, ND),
        in_specs=[
            pl.BlockSpec((TE, DFF), lambda e, d: (e, 0)),
            pl.BlockSpec((TE, 1), lambda e, d: (e, 0)),
            pl.BlockSpec((1, BD, DFF), lambda e, d: (e, d, 0)),
        ],
        out_specs=pl.BlockSpec((TE, BD), lambda e, d: (e, d)),
        out_shape=jax.ShapeDtypeStruct((T, D), jnp.float32),
        compiler_params=pltpu.CompilerParams(
            dimension_semantics=("parallel", "arbitrary")),
    )(h, gate2d, W2)


NW = 32
IPW = T // NW
TPW = (T // TOPK) // NW
G = 8
NG = TPW // G


@functools.partial(
    pl.kernel,
    out_type=jax.ShapeDtypeStruct((T,), jnp.int32),
    mesh=plsc.VectorSubcoreMesh(core_axis_name="c", subcore_axis_name="s"),
    scratch_types=[
        pltpu.VMEM((IPW,), jnp.int32),
        pltpu.VMEM((2, 128), jnp.int32),
        pltpu.VMEM((2, 128), jnp.int32),
    ],
)
def _build_inv(nidx_hbm, inv_hbm, nidx_v, idx2_v, val2_v):
    w = lax.axis_index("c") * 16 + lax.axis_index("s")
    base = w * IPW
    pltpu.sync_copy(nidx_hbm.at[pl.ds(base, IPW)], nidx_v)
    for i in range(IPW // 16):
        idx2_v[i // 8, pl.ds((i % 8) * 16, 16)] = nidx_v[pl.ds(i * 16, 16)]
        val2_v[i // 8, pl.ds((i % 8) * 16, 16)] = (
            jnp.arange(16, dtype=jnp.int32) + (base + i * 16))
    for j in range(2):
        pltpu.sync_copy(val2_v.at[j], inv_hbm.at[idx2_v.at[j]])


def _combine_body(res_hbm, inv_hbm, out_hbm, myinv_v, idx_g, rows_v,
                  obuf_v, sem):
    w = lax.axis_index("c") * 16 + lax.axis_index("s")
    t0 = w * TPW
    pltpu.sync_copy(inv_hbm.at[pl.ds(2 * t0, 2 * TPW)], myinv_v)
    for g in range(NG):
        idx_g[...] = myinv_v[pl.ds(g * 2 * G, 2 * G)]
        pltpu.async_copy(res_hbm.at[idx_g], rows_v, sem).wait()
        for j in range(G):
            def _add(k, _):
                obuf_v[j, pl.ds(k * 16, 16)] = (
                    rows_v[2 * j, pl.ds(k * 16, 16)]
                    + rows_v[2 * j + 1, pl.ds(k * 16, 16)])
                return 0
            lax.fori_loop(0, D // 16, _add, 0)
        pltpu.sync_copy(obuf_v, out_hbm.at[pl.ds(t0 + g * G, G)])


@functools.partial(
    pl.kernel,
    out_type=jax.ShapeDtypeStruct((T // TOPK, D), jnp.float32),
    mesh=plsc.VectorSubcoreMesh(core_axis_name="c", subcore_axis_name="s"),
    scratch_types=[
        pltpu.VMEM((2 * TPW,), jnp.int32),
        pltpu.VMEM((2 * G,), jnp.int32),
        pltpu.VMEM((2 * G, D), jnp.float32),
        pltpu.VMEM((G, D), jnp.float32),
        pltpu.SemaphoreType.DMA,
    ],
)
def _combine(res_hbm, inv_hbm, out_hbm, myinv_v, idx_g, rows_v,
             obuf_v, sem):
    _combine_body(res_hbm, inv_hbm, out_hbm, myinv_v, idx_g, rows_v,
                  obuf_v, sem)


def kernel(inputs_shard, gate_weight, choosed_experts, new_index, W1, W2):
    gate2d = gate_weight.reshape(T, 1)
    result = _expert_mlp(inputs_shard, gate2d, W1, W2)
    inv = _build_inv(new_index)
    out2 = _combine(result, inv)
    mlp_bias = jnp.zeros((D,), dtype=out2.dtype)
    return (out2, mlp_bias)
